# baseline (speedup 1.0000x reference)
import functools

import jax
import jax.numpy as jnp
from jax import lax
from jax.experimental import pallas as pl
from jax.experimental.pallas import tpu as pltpu

_BUDGET = 2048
_SEQ = 8192
_R = 64


def _norms_body(k_ref, norms_ref, valid_ref, count_ref):
    i = pl.program_id(0)
    x = k_ref[...]
    sq = jnp.sum(x * x, axis=-1)
    nr = jnp.sqrt(sq)
    nm = jnp.mean(nr, axis=-1)
    nz = jnp.any(x != 0, axis=-1)
    vd = jnp.any(nz, axis=-1)
    norms_ref[...] = nm
    valid_ref[...] = vd.astype(jnp.float32)
    cnt = jnp.sum(vd.astype(jnp.int32))

    @pl.when(i == 0)
    def _():
        count_ref[0, 0] = 0
    count_ref[0, 0] += cnt


def _norms_pass(k4):
    return pl.pallas_call(
        _norms_body,
        grid=(8,),
        in_specs=[pl.BlockSpec((8, 128, 16, 128), lambda i: (i, 0, 0, 0))],
        out_specs=[
            pl.BlockSpec((8, 128), lambda i: (i, 0)),
            pl.BlockSpec((8, 128), lambda i: (i, 0)),
            pl.BlockSpec(memory_space=pltpu.SMEM),
        ],
        out_shape=[
            jax.ShapeDtypeStruct((_R, 128), jnp.float32),
            jax.ShapeDtypeStruct((_R, 128), jnp.float32),
            jax.ShapeDtypeStruct((1, 1), jnp.int32),
        ],
    )(k4)


def _select_body(al_ref, norms_ref, valid_ref, fi_ref, n2_s, dest_s, sel_s):
    al = al_ref[0]
    norms = norms_ref[...]
    valid = valid_ref[...]
    r_i = lax.broadcasted_iota(jnp.int32, (_R, 128), 0)
    c_i = lax.broadcasted_iota(jnp.int32, (_R, 128), 1)
    gidx = r_i * 128 + c_i
    inf = jnp.float32(jnp.inf)
    n2 = jnp.where(valid > 0, norms, inf)
    n2 = jnp.where(gidx == 0, -inf, n2)
    n2 = jnp.where(gidx == al - 1, -inf, n2)
    gidx_f = gidx.astype(jnp.float32)
    n2_s[...] = n2
    lane = lax.broadcasted_iota(jnp.int32, (1, 128), 1)

    def rank_step(jc, acc):
        row = n2_s[pl.ds(jc, 1), :]
        jrow = (jc * 128 + lane).astype(jnp.float32)
        lt = (row[:, None, :] < n2[:, :, None])
        eq = (row[:, None, :] == n2[:, :, None]) & (
            jrow[:, None, :] < gidx_f[:, :, None])
        return acc + jnp.sum(lt.astype(jnp.float32) + eq.astype(jnp.float32),
                             axis=-1)

    rank = lax.fori_loop(0, _R, rank_step, jnp.zeros((_R, 128), jnp.float32))
    sel = (rank < _BUDGET).astype(jnp.float32)

    cA = lax.broadcasted_iota(jnp.int32, (128, 128), 0)
    cB = lax.broadcasted_iota(jnp.int32, (128, 128), 1)
    M = (cA <= cB).astype(jnp.float32)
    cs_in = jax.lax.dot(sel, M, preferred_element_type=jnp.float32)
    row_tot = jnp.sum(sel, axis=1, keepdims=True)
    rA = lax.broadcasted_iota(jnp.int32, (_R, _R), 0)
    rB = lax.broadcasted_iota(jnp.int32, (_R, _R), 1)
    S = (rB < rA).astype(jnp.float32)
    row_pre = jax.lax.dot(S, row_tot, preferred_element_type=jnp.float32)
    dest = cs_in - sel + row_pre
    dest_s[...] = jnp.where(sel > 0, dest, jnp.float32(-1.0))
    sel_s[...] = sel

    pr = lax.broadcasted_iota(jnp.int32, (16, 128), 0)
    pc = lax.broadcasted_iota(jnp.int32, (16, 128), 1)
    P = (pr * 128 + pc).astype(jnp.float32)

    def fi_step(rc, acc):
        d_row = dest_s[pl.ds(rc, 1), :]
        g_row = (rc * 128 + lane).astype(jnp.float32)
        hit = (d_row[:, None, :] == P[:, :, None])
        return acc + jnp.sum(hit.astype(jnp.float32) * g_row[:, None, :],
                             axis=-1)

    fi = lax.fori_loop(0, _R, fi_step, jnp.zeros((16, 128), jnp.float32))
    fi_ref[...] = fi.astype(jnp.int32)


def _select_pass(al, norms, valid):
    return pl.pallas_call(
        _select_body,
        in_specs=[
            pl.BlockSpec(memory_space=pltpu.SMEM),
            pl.BlockSpec((_R, 128), lambda: (0, 0)),
            pl.BlockSpec((_R, 128), lambda: (0, 0)),
        ],
        out_specs=pl.BlockSpec((16, 128), lambda: (0, 0)),
        out_shape=jax.ShapeDtypeStruct((16, 128), jnp.int32),
        scratch_shapes=[
            pltpu.VMEM((_R, 128), jnp.float32),
            pltpu.VMEM((_R, 128), jnp.float32),
            pltpu.VMEM((_R, 128), jnp.float32),
        ],
    )(al, norms, valid)


def kernel(q, k, v):
    seq = k.shape[0]
    k4 = k.reshape(_R, 128, 16, 128)
    norms, valid, al2 = _norms_pass(k4)
    al = al2.reshape((1,))
    actual_len = al[0]

    def do_compress():
        fi2 = _select_pass(al, norms, valid)
        fi = fi2.reshape(_BUDGET)
        comp_k = jnp.take(k, fi, axis=0)
        comp_v = jnp.take(v, fi, axis=0)
        kp = jnp.zeros_like(k).at[:_BUDGET].set(comp_k)
        vp = jnp.zeros_like(v).at[:_BUDGET].set(comp_v)
        return (kp, vp, jnp.array(_BUDGET, jnp.int32),
                actual_len.astype(jnp.int32))

    def do_nothing():
        return (k, v, actual_len.astype(jnp.int32),
                actual_len.astype(jnp.int32))

    return lax.cond(actual_len > _BUDGET, do_compress, do_nothing)
